# baseline (speedup 1.0000x reference)
import functools

import jax
import jax.numpy as jnp
from jax import lax
from jax.experimental import pallas as pl
from jax.experimental.pallas import tpu as pltpu
from jax.experimental.pallas import tpu_sc as plsc

B = 4
S = 4096
L = 16
NC = 2
NS = 16
NW = NC * NS
CPB = NW // B
CHUNK = S // CPB
VREGS = CHUNK // L
VOCAB_PAD = 1024


@functools.cache
def _build():
    mesh = plsc.VectorSubcoreMesh(core_axis_name="c", subcore_axis_name="s")

    @functools.partial(
        pl.kernel,
        mesh=mesh,
        out_type=jax.ShapeDtypeStruct((B, S), jnp.float32),
        scratch_types=[
            pltpu.VMEM((S,), jnp.int32),
            pltpu.VMEM((CHUNK,), jnp.float32),
            pltpu.VMEM((VOCAB_PAD,), jnp.int32),
        ],
        compiler_params=pltpu.CompilerParams(needs_layout_passes=False),
    )
    def counts_kernel(x_hbm, zeros_hbm, out_hbm, xv, outv, table):
        c = lax.axis_index("c")
        s = lax.axis_index("s")
        wid = s * NC + c
        b = wid // CPB
        ck = wid % CPB
        off = ck * CHUNK

        pltpu.sync_copy(x_hbm.at[b], xv)
        pltpu.sync_copy(zeros_hbm, table)

        ones = jnp.ones((L,), jnp.int32)

        def hist_body(i, carry):
            v = xv[pl.ds(i * L, L)]
            plsc.addupdate_scatter(table, [v], ones)
            return carry

        lax.fori_loop(0, ck * VREGS, hist_body, 0)

        def main_body(i, carry):
            v = xv[pl.ds(off + i * L, L)]
            cnt, _ = plsc.scan_count(v)
            base = plsc.load_gather(table, [v])
            new = base + cnt
            outv[pl.ds(i * L, L)] = new.astype(jnp.float32)
            plsc.addupdate_scatter(table, [v], ones)
            return carry

        lax.fori_loop(0, VREGS, main_body, 0)

        pltpu.sync_copy(outv, out_hbm.at[b, pl.ds(off, CHUNK)])

    return counts_kernel


def kernel(x):
    zeros = jnp.zeros((VOCAB_PAD,), jnp.int32)
    counts = _build()(x, zeros)
    return counts[..., None]


# speedup vs baseline: 1.0254x; 1.0254x over previous
import functools

import jax
import jax.numpy as jnp
from jax import lax
from jax.experimental import pallas as pl
from jax.experimental.pallas import tpu as pltpu
from jax.experimental.pallas import tpu_sc as plsc

B = 4
S = 4096
L = 16
NC = 2
NS = 16
NW = NC * NS
CPB = NW // B
CHUNK = S // CPB
VREGS = CHUNK // L
VOCAB_PAD = 1024


@functools.cache
def _build():
    mesh = plsc.VectorSubcoreMesh(core_axis_name="c", subcore_axis_name="s")

    @functools.partial(
        pl.kernel,
        mesh=mesh,
        out_type=jax.ShapeDtypeStruct((B, S), jnp.float32),
        scratch_types=[
            pltpu.VMEM((S,), jnp.int32),
            pltpu.VMEM((CHUNK,), jnp.float32),
            pltpu.VMEM((VOCAB_PAD,), jnp.int32),
        ],
        compiler_params=pltpu.CompilerParams(needs_layout_passes=False),
    )
    def counts_kernel(x_hbm, out_hbm, xv, outv, table):
        c = lax.axis_index("c")
        s = lax.axis_index("s")
        wid = s * NC + c
        b = wid // CPB
        ck = wid % CPB
        off = ck * CHUNK

        pltpu.sync_copy(x_hbm.at[b], xv)

        zeros = jnp.zeros((L,), jnp.int32)
        ones = jnp.ones((L,), jnp.int32)

        def zero_body(i, carry):
            table[pl.ds(i * L, L)] = zeros
            return carry

        lax.fori_loop(0, VOCAB_PAD // L, zero_body, 0)

        def hist_body(i, carry):
            for u in range(8):
                v = xv[pl.ds(i * (8 * L) + u * L, L)]
                plsc.addupdate_scatter(table, [v], ones)
            return carry

        lax.fori_loop(0, ck * (VREGS // 8), hist_body, 0)

        def main_body(i, carry):
            v = xv[pl.ds(off + i * L, L)]
            cnt, _ = plsc.scan_count(v)
            base = plsc.load_gather(table, [v])
            new = base + cnt
            outv[pl.ds(i * L, L)] = new.astype(jnp.float32)
            plsc.addupdate_scatter(table, [v], ones)
            return carry

        lax.fori_loop(0, VREGS, main_body, 0)

        pltpu.sync_copy(outv, out_hbm.at[b, pl.ds(off, CHUNK)])

    return counts_kernel


def kernel(x):
    counts = _build()(x)
    return counts[..., None]


# speedup vs baseline: 1.0610x; 1.0348x over previous
import functools

import jax
import jax.numpy as jnp
from jax import lax
from jax.experimental import pallas as pl
from jax.experimental.pallas import tpu as pltpu
from jax.experimental.pallas import tpu_sc as plsc

B = 4
S = 4096
L = 16
NC = 1
NS = 16
NW = NC * NS
CPB = NW // B
CHUNK = S // CPB
VREGS = CHUNK // L
VOCAB_PAD = 1024


@functools.cache
def _build():
    mesh = plsc.VectorSubcoreMesh(
        core_axis_name="c", subcore_axis_name="s", num_cores=NC
    )

    @functools.partial(
        pl.kernel,
        mesh=mesh,
        out_type=jax.ShapeDtypeStruct((B, S), jnp.float32),
        scratch_types=[
            pltpu.VMEM((S,), jnp.int32),
            pltpu.VMEM((CHUNK,), jnp.float32),
            pltpu.VMEM((VOCAB_PAD,), jnp.int32),
        ],
        compiler_params=pltpu.CompilerParams(needs_layout_passes=False),
    )
    def counts_kernel(x_hbm, out_hbm, xv, outv, table):
        c = lax.axis_index("c")
        s = lax.axis_index("s")
        wid = s * NC + c
        b = wid // CPB
        ck = wid % CPB
        off = ck * CHUNK

        pltpu.sync_copy(x_hbm.at[b], xv)

        zeros = jnp.zeros((L,), jnp.int32)
        ones = jnp.ones((L,), jnp.int32)

        def zero_body(i, carry):
            table[pl.ds(i * L, L)] = zeros
            return carry

        lax.fori_loop(0, VOCAB_PAD // L, zero_body, 0)

        def hist_body(i, carry):
            for u in range(8):
                v = xv[pl.ds(i * (8 * L) + u * L, L)]
                plsc.addupdate_scatter(table, [v], ones)
            return carry

        lax.fori_loop(0, ck * (VREGS // 8), hist_body, 0)

        def main_body(i, carry):
            v = xv[pl.ds(off + i * L, L)]
            cnt, _ = plsc.scan_count(v)
            base = plsc.load_gather(table, [v])
            new = base + cnt
            outv[pl.ds(i * L, L)] = new.astype(jnp.float32)
            plsc.addupdate_scatter(table, [v], ones)
            return carry

        lax.fori_loop(0, VREGS, main_body, 0)

        pltpu.sync_copy(outv, out_hbm.at[b, pl.ds(off, CHUNK)])

    return counts_kernel


def kernel(x):
    counts = _build()(x)
    return counts[..., None]
